# Initial kernel scaffold; baseline (speedup 1.0000x reference)
import jax
import jax.numpy as jnp
from jax.experimental import pallas as pl


def kernel(point_cloud, query_cloud, temperature):
    raise NotImplementedError("write your pallas kernel here")


# speedup vs baseline: 33.8756x; 33.8756x over previous
import functools

import jax
import jax.numpy as jnp
from jax.experimental import pallas as pl

K = 16


def _body(p_ref, q_ref, o_ref):
    p = p_ref[0]
    q = q_ref[0]
    px = p[0:1, :]
    py = p[1:2, :]
    pz = p[2:3, :]
    qx = q[:, 0:1]
    qy = q[:, 1:2]
    qz = q[:, 2:3]
    qq = qx * qx + qy * qy + qz * qz
    pp = px * px + py * py + pz * pz
    e = jax.lax.dot_general(
        q, p, (((1,), (0,)), ((), ())), preferred_element_type=jnp.float32
    )
    d2 = (qq + pp) - 2.0 * e

    d = d2
    m_first = None
    m_last = None
    for k in range(K):
        m = jnp.min(d, axis=1, keepdims=True)
        if k == 0:
            m_first = m
        m_last = m
        if k < K - 1:
            d = jnp.where(d == m, jnp.inf, d)

    w = jnp.where(d2 <= m_last, jnp.exp(m_first - d2), 0.0)
    den = jnp.sum(w, axis=1, keepdims=True)
    num = jax.lax.dot_general(
        w, p, (((1,), (1,)), ((), ())), preferred_element_type=jnp.float32
    )
    o_ref[0] = num / den


@jax.jit
def kernel(point_cloud, query_cloud, temperature):
    del temperature
    B, C, N = point_cloud.shape
    M = query_cloud.shape[2]
    Mb = 256

    qT = jnp.transpose(query_cloud, (0, 2, 1))

    out = pl.pallas_call(
        _body,
        grid=(B, M // Mb),
        in_specs=[
            pl.BlockSpec((1, C, N), lambda b, m: (b, 0, 0)),
            pl.BlockSpec((1, Mb, C), lambda b, m: (b, m, 0)),
        ],
        out_specs=pl.BlockSpec((1, Mb, C), lambda b, m: (b, m, 0)),
        out_shape=jax.ShapeDtypeStruct((B, M, C), jnp.float32),
    )(point_cloud, qT)

    return jnp.transpose(out, (0, 2, 1))


# speedup vs baseline: 48.3514x; 1.4273x over previous
import functools

import jax
import jax.numpy as jnp
from jax.experimental import pallas as pl

K = 16


def _body(p_ref, q_ref, o_ref):
    p = p_ref[0]
    q = q_ref[0]
    px = p[0:1, :]
    py = p[1:2, :]
    pz = p[2:3, :]
    qx = q[:, 0:1]
    qy = q[:, 1:2]
    qz = q[:, 2:3]
    qq = qx * qx + qy * qy + qz * qz
    pp = px * px + py * py + pz * pz
    e = jax.lax.dot_general(
        q, p, (((1,), (0,)), ((), ())), preferred_element_type=jnp.float32
    )
    d2 = (qq + pp) - 2.0 * e

    N = d2.shape[1]
    g = jnp.minimum(d2[:, : N // 2], d2[:, N // 2 :])
    g = jnp.minimum(g[:, : N // 4], g[:, N // 4 :])
    g = jnp.minimum(g[:, : N // 8], g[:, N // 8 :])

    m_first = None
    m_last = None
    for k in range(K):
        m = jnp.min(g, axis=1, keepdims=True)
        if k == 0:
            m_first = m
        m_last = m
        if k < K - 1:
            g = jnp.where(g == m, jnp.inf, g)

    seld = jnp.where(d2 <= m_last, d2, -jnp.inf)
    cnt = jnp.sum((d2 <= m_last).astype(jnp.float32), axis=1, keepdims=True)
    for _ in range(4):
        over = cnt > float(K)
        mx = jnp.max(seld, axis=1, keepdims=True)
        seld = jnp.where(over & (seld == mx), -jnp.inf, seld)
        cnt = cnt - jnp.where(over, 1.0, 0.0)

    w = jnp.exp(m_first - jnp.abs(seld))
    den = jnp.sum(w, axis=1, keepdims=True)
    num = jax.lax.dot_general(
        w, p, (((1,), (1,)), ((), ())), preferred_element_type=jnp.float32
    )
    o_ref[0] = num / den


@jax.jit
def kernel(point_cloud, query_cloud, temperature):
    del temperature
    B, C, N = point_cloud.shape
    M = query_cloud.shape[2]
    Mb = 256

    qT = jnp.transpose(query_cloud, (0, 2, 1))

    out = pl.pallas_call(
        _body,
        grid=(B, M // Mb),
        in_specs=[
            pl.BlockSpec((1, C, N), lambda b, m: (b, 0, 0)),
            pl.BlockSpec((1, Mb, C), lambda b, m: (b, m, 0)),
        ],
        out_specs=pl.BlockSpec((1, Mb, C), lambda b, m: (b, m, 0)),
        out_shape=jax.ShapeDtypeStruct((B, M, C), jnp.float32),
    )(point_cloud, qT)

    return jnp.transpose(out, (0, 2, 1))


# speedup vs baseline: 49.6102x; 1.0260x over previous
import functools

import jax
import jax.numpy as jnp
from jax.experimental import pallas as pl

K = 16


def _body(p_ref, q_ref, o_ref):
    p = p_ref[0]
    q = q_ref[0]
    px = p[0:1, :]
    py = p[1:2, :]
    pz = p[2:3, :]
    qx = q[:, 0:1]
    qy = q[:, 1:2]
    qz = q[:, 2:3]
    qq = qx * qx + qy * qy + qz * qz
    pp = px * px + py * py + pz * pz
    e = jax.lax.dot_general(
        q, p, (((1,), (0,)), ((), ())), preferred_element_type=jnp.float32
    )
    d2 = (qq + pp) - 2.0 * e

    N = d2.shape[1]
    g = jnp.minimum(d2[:, : N // 2], d2[:, N // 2 :])
    g = jnp.minimum(g[:, : N // 4], g[:, N // 4 :])
    g = jnp.minimum(g[:, : N // 8], g[:, N // 8 :])

    m_first = None
    m_last = None
    for k in range(K):
        m = jnp.min(g, axis=1, keepdims=True)
        if k == 0:
            m_first = m
        m_last = m
        if k < K - 1:
            g = jnp.where(g == m, jnp.inf, g)

    seld = jnp.where(d2 <= m_last, d2, -jnp.inf)
    cnt = jnp.sum((d2 <= m_last).astype(jnp.float32), axis=1, keepdims=True)
    for _ in range(4):
        over = cnt > float(K)
        mx = jnp.max(seld, axis=1, keepdims=True)
        seld = jnp.where(over & (seld == mx), -jnp.inf, seld)
        cnt = cnt - jnp.where(over, 1.0, 0.0)

    w = jnp.exp(m_first - jnp.abs(seld))
    den = jnp.sum(w, axis=1, keepdims=True)
    num = jax.lax.dot_general(
        w, p, (((1,), (1,)), ((), ())), preferred_element_type=jnp.float32
    )
    o_ref[0] = num / den


@jax.jit
def kernel(point_cloud, query_cloud, temperature):
    del temperature
    B, C, N = point_cloud.shape
    M = query_cloud.shape[2]
    Mb = 512

    qT = jnp.transpose(query_cloud, (0, 2, 1))

    out = pl.pallas_call(
        _body,
        grid=(B, M // Mb),
        in_specs=[
            pl.BlockSpec((1, C, N), lambda b, m: (b, 0, 0)),
            pl.BlockSpec((1, Mb, C), lambda b, m: (b, m, 0)),
        ],
        out_specs=pl.BlockSpec((1, Mb, C), lambda b, m: (b, m, 0)),
        out_shape=jax.ShapeDtypeStruct((B, M, C), jnp.float32),
    )(point_cloud, qT)

    return jnp.transpose(out, (0, 2, 1))


# speedup vs baseline: 85.3573x; 1.7206x over previous
import jax
import jax.numpy as jnp
from jax.experimental import pallas as pl

K = 16
PEEL = 6


def _body(p_ref, q_ref, o_ref):
    p = p_ref[0]
    q = q_ref[0]
    px = p[0:1, :]
    py = p[1:2, :]
    pz = p[2:3, :]
    qx = q[:, 0:1]
    qy = q[:, 1:2]
    qz = q[:, 2:3]
    qq = qx * qx + qy * qy + qz * qz
    pp = px * px + py * py + pz * pz
    e = jax.lax.dot_general(
        q, p, (((1,), (0,)), ((), ())), preferred_element_type=jnp.float32
    )
    d2 = (qq + pp) - 2.0 * e
    N = d2.shape[1]

    g = jnp.minimum(d2[:, : N // 2], d2[:, N // 2 :])
    for _ in range(4):
        half = g.shape[1] // 2
        g = jnp.minimum(g[:, :half], g[:, half:])

    m_first = None
    t = None
    for k in range(K):
        m = jnp.min(g, axis=1, keepdims=True)
        if k == 0:
            m_first = m
        t = m
        if k < K - 1:
            g = jnp.where(g == m, jnp.inf, g)

    cnt = jnp.sum((d2 <= t).astype(jnp.float32), axis=1, keepdims=True)

    lo = d2[:, : N // 2]
    hi = d2[:, N // 2 :]
    h = jnp.maximum(
        jnp.where(lo <= t, lo, -jnp.inf), jnp.where(hi <= t, hi, -jnp.inf)
    )
    for _ in range(4):
        half = h.shape[1] // 2
        h = jnp.maximum(h[:, :half], h[:, half:])

    v = []
    for j in range(PEEL):
        mx = jnp.max(h, axis=1, keepdims=True)
        v.append(mx)
        if j < PEEL - 1:
            h = jnp.where(h == mx, -jnp.inf, h)
    r_cut = jnp.full_like(cnt, jnp.inf)
    for j in range(PEEL):
        r_cut = jnp.where(cnt >= float(K + j + 1), v[j], r_cut)

    w = jnp.where(
        (d2 <= t) & (d2 < r_cut), jnp.exp(m_first - d2), 0.0
    )
    ones = jnp.ones((1, N), dtype=jnp.float32)
    pe = jnp.concatenate([p, ones], axis=0)
    acc = jax.lax.dot_general(
        w, pe, (((1,), (1,)), ((), ())), preferred_element_type=jnp.float32
    )
    o_ref[0] = acc[:, 0:3] / acc[:, 3:4]


@jax.jit
def kernel(point_cloud, query_cloud, temperature):
    del temperature
    B, C, N = point_cloud.shape
    M = query_cloud.shape[2]
    Mb = 512

    qT = jnp.transpose(query_cloud, (0, 2, 1))

    out = pl.pallas_call(
        _body,
        grid=(B, M // Mb),
        in_specs=[
            pl.BlockSpec((1, C, N), lambda b, m: (b, 0, 0)),
            pl.BlockSpec((1, Mb, C), lambda b, m: (b, m, 0)),
        ],
        out_specs=pl.BlockSpec((1, Mb, C), lambda b, m: (b, m, 0)),
        out_shape=jax.ShapeDtypeStruct((B, M, C), jnp.float32),
    )(point_cloud, qT)

    return jnp.transpose(out, (0, 2, 1))


# speedup vs baseline: 85.3716x; 1.0002x over previous
import jax
import jax.numpy as jnp
from jax.experimental import pallas as pl

K = 16
PEEL = 6


def _body(p_ref, q_ref, o_ref):
    p = p_ref[0]
    q = q_ref[0]
    px = p[0:1, :]
    py = p[1:2, :]
    pz = p[2:3, :]
    qx = q[:, 0:1]
    qy = q[:, 1:2]
    qz = q[:, 2:3]
    qq = qx * qx + qy * qy + qz * qz
    pp = px * px + py * py + pz * pz
    e = jax.lax.dot_general(
        q, p, (((1,), (0,)), ((), ())), preferred_element_type=jnp.float32
    )
    N = e.shape[1]

    def d2s(lo, hi):
        return (qq + pp[:, lo:hi]) - 2.0 * e[:, lo:hi]

    g = jnp.minimum(d2s(0, N // 2), d2s(N // 2, N))
    for _ in range(4):
        half = g.shape[1] // 2
        g = jnp.minimum(g[:, :half], g[:, half:])

    m_first = None
    t = None
    for k in range(K):
        m = jnp.min(g, axis=1, keepdims=True)
        if k == 0:
            m_first = m
        t = m
        if k < K - 1:
            g = jnp.where(g == m, jnp.inf, g)

    cnt = jnp.sum(
        (d2s(0, N // 2) <= t).astype(jnp.float32), axis=1, keepdims=True
    ) + jnp.sum(
        (d2s(N // 2, N) <= t).astype(jnp.float32), axis=1, keepdims=True
    )

    lo = d2s(0, N // 2)
    hi = d2s(N // 2, N)
    h = jnp.maximum(
        jnp.where(lo <= t, lo, -jnp.inf), jnp.where(hi <= t, hi, -jnp.inf)
    )
    for _ in range(4):
        half = h.shape[1] // 2
        h = jnp.maximum(h[:, :half], h[:, half:])

    v = []
    for j in range(PEEL):
        mx = jnp.max(h, axis=1, keepdims=True)
        v.append(mx)
        if j < PEEL - 1:
            h = jnp.where(h == mx, -jnp.inf, h)
    r_cut = jnp.full_like(cnt, jnp.inf)
    for j in range(PEEL):
        r_cut = jnp.where(cnt >= float(K + j + 1), v[j], r_cut)

    d2f = d2s(0, N)
    w = jnp.where(
        (d2f <= t) & (d2f < r_cut), jnp.exp(m_first - d2f), 0.0
    )
    ones = jnp.ones((1, N), dtype=jnp.float32)
    pe = jnp.concatenate([p, ones], axis=0)
    acc = jax.lax.dot_general(
        w, pe, (((1,), (1,)), ((), ())), preferred_element_type=jnp.float32
    )
    o_ref[0] = acc[:, 0:3] / acc[:, 3:4]


@jax.jit
def kernel(point_cloud, query_cloud, temperature):
    del temperature
    B, C, N = point_cloud.shape
    M = query_cloud.shape[2]
    Mb = 512

    qT = jnp.transpose(query_cloud, (0, 2, 1))

    out = pl.pallas_call(
        _body,
        grid=(B, M // Mb),
        in_specs=[
            pl.BlockSpec((1, C, N), lambda b, m: (b, 0, 0)),
            pl.BlockSpec((1, Mb, C), lambda b, m: (b, m, 0)),
        ],
        out_specs=pl.BlockSpec((1, Mb, C), lambda b, m: (b, m, 0)),
        out_shape=jax.ShapeDtypeStruct((B, M, C), jnp.float32),
    )(point_cloud, qT)

    return jnp.transpose(out, (0, 2, 1))


# speedup vs baseline: 86.2985x; 1.0109x over previous
import jax
import jax.numpy as jnp
from jax.experimental import pallas as pl

K = 16
PEEL = 8


def _body(p_ref, q_ref, o_ref):
    p = p_ref[0]
    q = q_ref[0]
    px = p[0:1, :]
    py = p[1:2, :]
    pz = p[2:3, :]
    qx = q[:, 0:1]
    qy = q[:, 1:2]
    qz = q[:, 2:3]
    qq = qx * qx + qy * qy + qz * qz
    pp = px * px + py * py + pz * pz
    e = jax.lax.dot_general(
        q, p, (((1,), (0,)), ((), ())), preferred_element_type=jnp.float32
    )
    N = e.shape[1]

    def d2s(lo, hi):
        return (qq + pp[:, lo:hi]) - 2.0 * e[:, lo:hi]

    g = jnp.minimum(d2s(0, N // 2), d2s(N // 2, N))
    for _ in range(5):
        half = g.shape[1] // 2
        g = jnp.minimum(g[:, :half], g[:, half:])

    m_first = None
    t = None
    for k in range(K):
        m = jnp.min(g, axis=1, keepdims=True)
        if k == 0:
            m_first = m
        t = m
        if k < K - 1:
            g = jnp.where(g == m, jnp.inf, g)

    cnt = jnp.sum(
        (d2s(0, N // 2) <= t).astype(jnp.float32), axis=1, keepdims=True
    ) + jnp.sum(
        (d2s(N // 2, N) <= t).astype(jnp.float32), axis=1, keepdims=True
    )

    lo = d2s(0, N // 2)
    hi = d2s(N // 2, N)
    h = jnp.maximum(
        jnp.where(lo <= t, lo, -jnp.inf), jnp.where(hi <= t, hi, -jnp.inf)
    )
    for _ in range(5):
        half = h.shape[1] // 2
        h = jnp.maximum(h[:, :half], h[:, half:])

    v = []
    for j in range(PEEL):
        mx = jnp.max(h, axis=1, keepdims=True)
        v.append(mx)
        if j < PEEL - 1:
            h = jnp.where(h == mx, -jnp.inf, h)
    r_cut = jnp.full_like(cnt, jnp.inf)
    for j in range(PEEL):
        r_cut = jnp.where(cnt >= float(K + j + 1), v[j], r_cut)

    d2f = d2s(0, N)
    w = jnp.where(
        (d2f <= t) & (d2f < r_cut), jnp.exp(m_first - d2f), 0.0
    )
    ones = jnp.ones((1, N), dtype=jnp.float32)
    pe = jnp.concatenate([p, ones], axis=0)
    acc = jax.lax.dot_general(
        w, pe, (((1,), (1,)), ((), ())), preferred_element_type=jnp.float32
    )
    o_ref[0] = acc[:, 0:3] / acc[:, 3:4]


@jax.jit
def kernel(point_cloud, query_cloud, temperature):
    del temperature
    B, C, N = point_cloud.shape
    M = query_cloud.shape[2]
    Mb = 512

    qT = jnp.transpose(query_cloud, (0, 2, 1))

    out = pl.pallas_call(
        _body,
        grid=(B, M // Mb),
        in_specs=[
            pl.BlockSpec((1, C, N), lambda b, m: (b, 0, 0)),
            pl.BlockSpec((1, Mb, C), lambda b, m: (b, m, 0)),
        ],
        out_specs=pl.BlockSpec((1, Mb, C), lambda b, m: (b, m, 0)),
        out_shape=jax.ShapeDtypeStruct((B, M, C), jnp.float32),
    )(point_cloud, qT)

    return jnp.transpose(out, (0, 2, 1))


# speedup vs baseline: 88.0419x; 1.0202x over previous
import jax
import jax.numpy as jnp
from jax.experimental import pallas as pl

K = 16
PEEL = 8


def _body(p_ref, q_ref, o_ref):
    p = p_ref[0]
    q = q_ref[0]
    px = p[0:1, :]
    py = p[1:2, :]
    pz = p[2:3, :]
    qx = q[:, 0:1]
    qy = q[:, 1:2]
    qz = q[:, 2:3]
    qq = qx * qx + qy * qy + qz * qz
    pp = px * px + py * py + pz * pz
    e = jax.lax.dot_general(
        q, p, (((1,), (0,)), ((), ())), preferred_element_type=jnp.float32
    )
    N = e.shape[1]

    def d2s(lo, hi):
        return (qq + pp[:, lo:hi]) - 2.0 * e[:, lo:hi]

    g = jnp.minimum(d2s(0, N // 2), d2s(N // 2, N))
    for _ in range(5):
        half = g.shape[1] // 2
        g = jnp.minimum(g[:, :half], g[:, half:])

    m_first = None
    t = None
    for k in range(K):
        m = jnp.min(g, axis=1, keepdims=True)
        if k == 0:
            m_first = m
        t = m
        if k < K - 1:
            g = jnp.where(g == m, jnp.inf, g)

    cnt = jnp.sum(
        (d2s(0, N // 2) <= t).astype(jnp.float32), axis=1, keepdims=True
    ) + jnp.sum(
        (d2s(N // 2, N) <= t).astype(jnp.float32), axis=1, keepdims=True
    )

    lo = d2s(0, N // 2)
    hi = d2s(N // 2, N)
    h = jnp.maximum(
        jnp.where(lo <= t, lo, -jnp.inf), jnp.where(hi <= t, hi, -jnp.inf)
    )
    for _ in range(5):
        half = h.shape[1] // 2
        h = jnp.maximum(h[:, :half], h[:, half:])

    v = []
    for j in range(PEEL):
        mx = jnp.max(h, axis=1, keepdims=True)
        v.append(mx)
        if j < PEEL - 1:
            h = jnp.where(h == mx, -jnp.inf, h)
    r_cut = jnp.full_like(cnt, jnp.inf)
    for j in range(PEEL):
        r_cut = jnp.where(cnt >= float(K + j + 1), v[j], r_cut)

    d2f = d2s(0, N)
    w = jnp.where(
        (d2f <= t) & (d2f < r_cut), jnp.exp(m_first - d2f), 0.0
    )
    ones = jnp.ones((1, N), dtype=jnp.float32)
    pe = jnp.concatenate([p, ones], axis=0)
    acc = jax.lax.dot_general(
        w, pe, (((1,), (1,)), ((), ())), preferred_element_type=jnp.float32
    )
    o_ref[0] = acc[:, 0:3] / acc[:, 3:4]


@jax.jit
def kernel(point_cloud, query_cloud, temperature):
    del temperature
    B, C, N = point_cloud.shape
    M = query_cloud.shape[2]
    Mb = 1024

    qT = jnp.transpose(query_cloud, (0, 2, 1))

    out = pl.pallas_call(
        _body,
        grid=(B, M // Mb),
        in_specs=[
            pl.BlockSpec((1, C, N), lambda b, m: (b, 0, 0)),
            pl.BlockSpec((1, Mb, C), lambda b, m: (b, m, 0)),
        ],
        out_specs=pl.BlockSpec((1, Mb, C), lambda b, m: (b, m, 0)),
        out_shape=jax.ShapeDtypeStruct((B, M, C), jnp.float32),
    )(point_cloud, qT)

    return jnp.transpose(out, (0, 2, 1))


# speedup vs baseline: 88.1237x; 1.0009x over previous
import jax
import jax.numpy as jnp
from jax.experimental import pallas as pl

K = 16
PEEL = 8


def _body(p_ref, q_ref, o_ref):
    p = p_ref[0]
    q = q_ref[0]
    px = p[0:1, :]
    py = p[1:2, :]
    pz = p[2:3, :]
    qx = q[:, 0:1]
    qy = q[:, 1:2]
    qz = q[:, 2:3]
    qq = qx * qx + qy * qy + qz * qz
    pp = px * px + py * py + pz * pz
    e = jax.lax.dot_general(
        q, p, (((1,), (0,)), ((), ())), preferred_element_type=jnp.float32
    )
    N = e.shape[1]

    def d2s(lo, hi):
        return (qq + pp[:, lo:hi]) - 2.0 * e[:, lo:hi]

    g = jnp.minimum(d2s(0, N // 2), d2s(N // 2, N))
    for _ in range(5):
        half = g.shape[1] // 2
        g = jnp.minimum(g[:, :half], g[:, half:])

    m_first = None
    t = None
    for k in range(K):
        m = jnp.min(g, axis=1, keepdims=True)
        if k == 0:
            m_first = m
        t = m
        if k < K - 1:
            g = jnp.where(g == m, jnp.inf, g)

    cnt = jnp.sum(
        (d2s(0, N // 2) <= t).astype(jnp.float32), axis=1, keepdims=True
    ) + jnp.sum(
        (d2s(N // 2, N) <= t).astype(jnp.float32), axis=1, keepdims=True
    )

    lo = d2s(0, N // 2)
    hi = d2s(N // 2, N)
    h = jnp.maximum(
        jnp.where(lo <= t, lo, -jnp.inf), jnp.where(hi <= t, hi, -jnp.inf)
    )
    for _ in range(5):
        half = h.shape[1] // 2
        h = jnp.maximum(h[:, :half], h[:, half:])

    v = []
    for j in range(PEEL):
        mx = jnp.max(h, axis=1, keepdims=True)
        v.append(mx)
        if j < PEEL - 1:
            h = jnp.where(h == mx, -jnp.inf, h)
    r_cut = jnp.full_like(cnt, jnp.inf)
    for j in range(PEEL):
        r_cut = jnp.where(cnt >= float(K + j + 1), v[j], r_cut)

    d2f = d2s(0, N)
    w = jnp.where(
        (d2f <= t) & (d2f < r_cut), jnp.exp(m_first - d2f), 0.0
    ).astype(jnp.bfloat16)
    ones = jnp.ones((1, N), dtype=jnp.float32)
    pe = jnp.concatenate([p, ones], axis=0).astype(jnp.bfloat16)
    acc = jax.lax.dot_general(
        w, pe, (((1,), (1,)), ((), ())), preferred_element_type=jnp.float32
    )
    o_ref[0] = acc[:, 0:3] / acc[:, 3:4]


@jax.jit
def kernel(point_cloud, query_cloud, temperature):
    del temperature
    B, C, N = point_cloud.shape
    M = query_cloud.shape[2]
    Mb = 1024

    qT = jnp.transpose(query_cloud, (0, 2, 1))

    out = pl.pallas_call(
        _body,
        grid=(B, M // Mb),
        in_specs=[
            pl.BlockSpec((1, C, N), lambda b, m: (b, 0, 0)),
            pl.BlockSpec((1, Mb, C), lambda b, m: (b, m, 0)),
        ],
        out_specs=pl.BlockSpec((1, Mb, C), lambda b, m: (b, m, 0)),
        out_shape=jax.ShapeDtypeStruct((B, M, C), jnp.float32),
    )(point_cloud, qT)

    return jnp.transpose(out, (0, 2, 1))
